# baseline (speedup 1.0000x reference)
import functools

import jax
import jax.numpy as jnp
from jax import lax
from jax.experimental import pallas as pl
from jax.experimental.pallas import tpu as pltpu
from jax.experimental.pallas import tpu_sc as plsc

_VOCAB = 1_000_000
_DIM = 64
_BATCH = 4096
_SEQ = 200


_BR = 8000


def _matvec_body(t_ref, w_ref, o_ref):
    o_ref[...] = jnp.sum(t_ref[...] * w_ref[...], axis=1, keepdims=True)


def _project_table(table, w_scaled):
    return pl.pallas_call(
        _matvec_body,
        grid=(_VOCAB // _BR,),
        in_specs=[
            pl.BlockSpec((_BR, _DIM), lambda i: (i, 0)),
            pl.BlockSpec((1, _DIM), lambda i: (0, 0)),
        ],
        out_specs=pl.BlockSpec((_BR, 1), lambda i: (i, 0)),
        out_shape=jax.ShapeDtypeStruct((_VOCAB, 1), jnp.float32),
    )(table, w_scaled)


_NC = 2
_NS = 16
_NW = _NC * _NS
_ROWS_W = _BATCH // _NW
_GROUPS = _ROWS_W // 16
_IPW = _ROWS_W * _SEQ
_SC_UNROLL = 8


def _sc_body(idx_hbm, p_hbm, b_hbm, out_hbm, idx_v, vals_v, out_v, b_v, sem):
    wid = lax.axis_index("s") * _NC + lax.axis_index("c")
    pltpu.sync_copy(idx_hbm.at[wid], idx_v)
    pltpu.sync_copy(b_hbm, b_v)
    pltpu.async_copy(p_hbm.at[idx_v], vals_v, sem).wait()
    bias = b_v[...]
    for g in range(_GROUPS):
        base = g * (16 * _SEQ)

        def body(t, acc, base=base):
            off = base + t * (16 * _SC_UNROLL)
            for k in range(_SC_UNROLL):
                acc = acc + vals_v[pl.ds(off + k * 16, 16)]
            return acc

        acc = lax.fori_loop(0, _SEQ // _SC_UNROLL, body, bias)
        out_v[pl.ds(g * 16, 16)] = acc
    pltpu.sync_copy(out_v, out_hbm.at[pl.ds(wid * _ROWS_W, _ROWS_W)])


@functools.lru_cache(maxsize=1)
def _sc_gather_sum():
    return pl.kernel(
        _sc_body,
        out_type=jax.ShapeDtypeStruct((_BATCH,), jnp.float32),
        mesh=plsc.VectorSubcoreMesh(
            core_axis_name="c", subcore_axis_name="s", num_cores=_NC, num_subcores=_NS
        ),
        scratch_types=[
            pltpu.VMEM((_IPW,), jnp.int32),
            pltpu.VMEM((_IPW,), jnp.float32),
            pltpu.VMEM((_ROWS_W,), jnp.float32),
            pltpu.VMEM((16,), jnp.float32),
            pltpu.SemaphoreType.DMA,
        ],
    )


def kernel(idx_tensor, table, fc_w, fc_b):
    w_scaled = fc_w.astype(jnp.float32) * (1.0 / _SEQ)
    p = _project_table(table, w_scaled).reshape(_VOCAB)
    idx_il = (
        idx_tensor.reshape(_NW, _GROUPS, 16, _SEQ)
        .transpose(0, 1, 3, 2)
        .reshape(_NW, _IPW)
    )
    b16 = jnp.broadcast_to(fc_b.astype(jnp.float32), (16,))
    return _sc_gather_sum()(idx_il, p, b16)


# speedup vs baseline: 1.0086x; 1.0086x over previous
import functools

import jax
import jax.numpy as jnp
from jax import lax
from jax.experimental import pallas as pl
from jax.experimental.pallas import tpu as pltpu
from jax.experimental.pallas import tpu_sc as plsc

_VOCAB = 1_000_000
_DIM = 64
_BATCH = 4096
_SEQ = 200


_BR = 8000


def _matvec_body(t_ref, w_ref, o_ref):
    o_ref[...] = jnp.dot(t_ref[...], w_ref[...], preferred_element_type=jnp.float32)


def _project_table(table, w_scaled):
    return pl.pallas_call(
        _matvec_body,
        grid=(_VOCAB // _BR,),
        in_specs=[
            pl.BlockSpec((_BR, _DIM), lambda i: (i, 0)),
            pl.BlockSpec((_DIM, 1), lambda i: (0, 0)),
        ],
        out_specs=pl.BlockSpec((_BR, 1), lambda i: (i, 0)),
        out_shape=jax.ShapeDtypeStruct((_VOCAB, 1), jnp.float32),
    )(table, w_scaled)


_NC = 2
_NS = 16
_NW = _NC * _NS
_ROWS_W = _BATCH // _NW
_GROUPS = _ROWS_W // 16
_IPW = _ROWS_W * _SEQ
_SC_UNROLL = 8


def _sc_body(idx_hbm, p_hbm, b_hbm, out_hbm, idx_v, vals_v, out_v, b_v, sem):
    wid = lax.axis_index("s") * _NC + lax.axis_index("c")
    pltpu.sync_copy(idx_hbm.at[wid], idx_v)
    pltpu.sync_copy(b_hbm, b_v)
    pltpu.async_copy(p_hbm.at[idx_v], vals_v, sem).wait()
    bias = b_v[...]
    for g in range(_GROUPS):
        base = g * (16 * _SEQ)

        def body(t, acc, base=base):
            off = base + t * (16 * _SC_UNROLL)
            for k in range(_SC_UNROLL):
                acc = acc + vals_v[pl.ds(off + k * 16, 16)]
            return acc

        acc = lax.fori_loop(0, _SEQ // _SC_UNROLL, body, bias)
        out_v[pl.ds(g * 16, 16)] = acc
    pltpu.sync_copy(out_v, out_hbm.at[pl.ds(wid * _ROWS_W, _ROWS_W)])


@functools.lru_cache(maxsize=1)
def _sc_gather_sum():
    return pl.kernel(
        _sc_body,
        out_type=jax.ShapeDtypeStruct((_BATCH,), jnp.float32),
        mesh=plsc.VectorSubcoreMesh(
            core_axis_name="c", subcore_axis_name="s", num_cores=_NC, num_subcores=_NS
        ),
        scratch_types=[
            pltpu.VMEM((_IPW,), jnp.int32),
            pltpu.VMEM((_IPW,), jnp.float32),
            pltpu.VMEM((_ROWS_W,), jnp.float32),
            pltpu.VMEM((16,), jnp.float32),
            pltpu.SemaphoreType.DMA,
        ],
    )


def kernel(idx_tensor, table, fc_w, fc_b):
    w_scaled = fc_w.astype(jnp.float32).reshape(_DIM, 1) * (1.0 / _SEQ)
    p = _project_table(table, w_scaled).reshape(_VOCAB)
    idx_il = (
        idx_tensor.reshape(_NW, _GROUPS, 16, _SEQ)
        .transpose(0, 1, 3, 2)
        .reshape(_NW, _IPW)
    )
    b16 = jnp.broadcast_to(fc_b.astype(jnp.float32), (16,))
    return _sc_gather_sum()(idx_il, p, b16)
